# baseline (speedup 1.0000x reference)
import functools

import jax
import jax.numpy as jnp
import numpy as np
from jax import lax
from jax.experimental import pallas as pl
from jax.experimental.pallas import tpu as pltpu
from jax.experimental.pallas import tpu_sc as plsc

N = 16384
C = 16
RNA_EMBED = 128
ATOM_EMBED = 64
NUM_RNA_TYPE = 8
NUM_ATOM_TYPE = 64
NUM_ATOM_POS = 64
HIST = NUM_ATOM_TYPE + NUM_ATOM_POS
OUT_D = RNA_EMBED + ATOM_EMBED
EPS = 1e-10
BLK = 2048

NUM_SC = 2
NUM_SUBCORES = 16
NW = NUM_SC * NUM_SUBCORES
CHUNK = N // NW


def _hist_body(a_hbm, ap_hbm, hist_hbm, a_v, ap_v, hist_v):
    wid = lax.axis_index("s") * NUM_SC + lax.axis_index("c")
    base = wid * CHUNK
    pltpu.sync_copy(a_hbm.at[pl.ds(base * C, CHUNK * C)], a_v)
    pltpu.sync_copy(ap_hbm.at[pl.ds(base * C, CHUNK * C)], ap_v)

    def row(n, _):
        a_vec = a_v[pl.ds(n * C, 16)]
        ap_vec = ap_v[pl.ds(n * C, 16)]
        m = jnp.where(ap_vec != 0, 1.0, 0.0).astype(jnp.float32)
        for k in range(HIST // 16):
            hist_v[pl.ds(n * HIST + 16 * k, 16)] = jnp.zeros((16,), jnp.float32)
        flat_base = jnp.full((16,), n * HIST, jnp.int32)
        plsc.addupdate_scatter(hist_v, [flat_base + a_vec], m)
        plsc.addupdate_scatter(
            hist_v, [flat_base + (ap_vec + NUM_ATOM_TYPE)], m)
        return 0

    lax.fori_loop(0, CHUNK, row, 0)
    pltpu.sync_copy(hist_v, hist_hbm.at[pl.ds(base * HIST, CHUNK * HIST)])


_hist_kernel = functools.partial(
    pl.kernel,
    mesh=plsc.VectorSubcoreMesh(core_axis_name="c", subcore_axis_name="s"),
    out_type=jax.ShapeDtypeStruct((N * HIST,), jnp.float32),
    compiler_params=pltpu.CompilerParams(needs_layout_passes=False),
    scratch_types=[
        pltpu.VMEM((CHUNK * C,), jnp.int32),
        pltpu.VMEM((CHUNK * C,), jnp.int32),
        pltpu.VMEM((CHUNK * HIST,), jnp.float32),
    ],
)(_hist_body)


def _rna_body(s_ref, rp_ref, sm_ref, rna_t_ref, mod_t_ref, fp_ref, out_ref):
    b = BLK
    s = s_ref[0, 0, :]
    sm = sm_ref[0, 0, :]
    pos = rp_ref[0, 0, :].astype(jnp.float32)

    iota8 = lax.broadcasted_iota(jnp.int32, (b, NUM_RNA_TYPE), 1)
    oh_s = (s[:, None] == iota8).astype(jnp.float32)
    rna = jnp.dot(oh_s, rna_t_ref[...], preferred_element_type=jnp.float32)
    iota3 = lax.broadcasted_iota(jnp.int32, (b, 3), 1)
    oh_m = (sm[:, None] == iota3).astype(jnp.float32)
    rna = rna + jnp.dot(oh_m, mod_t_ref[...], preferred_element_type=jnp.float32)
    ang = pos[:, None] * fp_ref[0:1, :] + fp_ref[1:2, :]
    out_ref[...] = rna + jnp.sin(ang)


def _assemble_body(rna_ref, hist_ref, combo_ref, out_ref):
    pooled = jnp.dot(hist_ref[...], combo_ref[...],
                     preferred_element_type=jnp.float32)
    denom = pooled[:, ATOM_EMBED:ATOM_EMBED + 1]
    out_ref[:, 0:RNA_EMBED] = rna_ref[...]
    out_ref[:, RNA_EMBED:OUT_D] = (
        pooled[:, 0:ATOM_EMBED] * (1.0 / (denom + EPS)))


@jax.jit
def _run(S, RP, A, AP, SM, rna_table, atom_table, atom_pos_table, mod_table):
    hist = _hist_kernel(A.astype(jnp.int32).reshape(N * C),
                        AP.astype(jnp.int32).reshape(N * C)).reshape(N, HIST)

    combo = jnp.zeros((HIST, 128), jnp.float32)
    combo = combo.at[0:NUM_ATOM_TYPE, 0:ATOM_EMBED].set(atom_table)
    combo = combo.at[NUM_ATOM_TYPE:HIST, 0:ATOM_EMBED].set(atom_pos_table)
    combo = combo.at[0:NUM_ATOM_TYPE, ATOM_EMBED].set(1.0)

    d = np.arange(RNA_EMBED)
    freq_np = np.power(10000.0, -2.0 * (d // 2) / RNA_EMBED)
    phase_np = np.where(d % 2 == 0, 0.0, np.pi / 2)
    fp = jnp.asarray(np.stack([freq_np, phase_np]), jnp.float32)

    nb = N // BLK
    s3 = S.reshape(nb, 1, BLK).astype(jnp.int32)
    rp3 = RP.reshape(nb, 1, BLK).astype(jnp.int32)
    sm3 = SM.reshape(nb, 1, BLK).astype(jnp.int32)
    vec_spec = pl.BlockSpec((1, 1, BLK), lambda i: (i, 0, 0))

    rna = pl.pallas_call(
        _rna_body,
        grid=(nb,),
        in_specs=[
            vec_spec, vec_spec, vec_spec,
            pl.BlockSpec((NUM_RNA_TYPE, RNA_EMBED), lambda i: (0, 0)),
            pl.BlockSpec((3, RNA_EMBED), lambda i: (0, 0)),
            pl.BlockSpec((2, RNA_EMBED), lambda i: (0, 0)),
        ],
        out_specs=pl.BlockSpec((BLK, RNA_EMBED), lambda i: (i, 0)),
        out_shape=jax.ShapeDtypeStruct((N, RNA_EMBED), jnp.float32),
    )(s3, rp3, sm3, rna_table, mod_table, fp)

    return pl.pallas_call(
        _assemble_body,
        grid=(nb,),
        in_specs=[
            pl.BlockSpec((BLK, RNA_EMBED), lambda i: (i, 0)),
            pl.BlockSpec((BLK, HIST), lambda i: (i, 0)),
            pl.BlockSpec((HIST, 128), lambda i: (0, 0)),
        ],
        out_specs=pl.BlockSpec((BLK, OUT_D), lambda i: (i, 0)),
        out_shape=jax.ShapeDtypeStruct((N, OUT_D), jnp.float32),
    )(rna, hist, combo)


def kernel(S, RP, A, AP, SM, rna_table, atom_table, atom_pos_table, mod_table):
    return _run(S, RP, A, AP, SM, rna_table, atom_table,
                atom_pos_table, mod_table)


# speedup vs baseline: 1.3291x; 1.3291x over previous
import functools

import jax
import jax.numpy as jnp
import numpy as np
from jax import lax
from jax.experimental import pallas as pl
from jax.experimental.pallas import tpu as pltpu
from jax.experimental.pallas import tpu_sc as plsc

N = 16384
C = 16
RNA_EMBED = 128
ATOM_EMBED = 64
NUM_RNA_TYPE = 8
NUM_ATOM_TYPE = 64
NUM_ATOM_POS = 64
HIST = NUM_ATOM_TYPE + NUM_ATOM_POS
OUT_D = RNA_EMBED + ATOM_EMBED
EPS = 1e-10
BLK = 2048

NUM_SC = 2
NUM_SUBCORES = 16
NW = NUM_SC * NUM_SUBCORES
CHUNK = N // NW


def _hist_body(a_hbm, ap_hbm, hist_hbm, a_v, ap_v, hist_v):
    wid = lax.axis_index("s") * NUM_SC + lax.axis_index("c")
    base = wid * CHUNK
    pltpu.sync_copy(a_hbm.at[pl.ds(base * C, CHUNK * C)], a_v)
    pltpu.sync_copy(ap_hbm.at[pl.ds(base * C, CHUNK * C)], ap_v)

    def row(n, _):
        a_vec = a_v[pl.ds(n * C, 16)]
        ap_vec = ap_v[pl.ds(n * C, 16)]
        m = jnp.where(ap_vec != 0, 1.0, 0.0).astype(jnp.float32)
        for k in range(HIST // 16):
            hist_v[pl.ds(n * HIST + 16 * k, 16)] = jnp.zeros((16,), jnp.float32)
        flat_base = jnp.full((16,), n * HIST, jnp.int32)
        plsc.addupdate_scatter(hist_v, [flat_base + a_vec], m)
        plsc.addupdate_scatter(
            hist_v, [flat_base + (ap_vec + NUM_ATOM_TYPE)], m)
        return 0

    lax.fori_loop(0, CHUNK, row, 0)
    pltpu.sync_copy(hist_v, hist_hbm.at[pl.ds(base * HIST, CHUNK * HIST)])


_hist_kernel = functools.partial(
    pl.kernel,
    mesh=plsc.VectorSubcoreMesh(core_axis_name="c", subcore_axis_name="s"),
    out_type=jax.ShapeDtypeStruct((N * HIST,), jnp.float32),
    compiler_params=pltpu.CompilerParams(needs_layout_passes=False),
    scratch_types=[
        pltpu.VMEM((CHUNK * C,), jnp.int32),
        pltpu.VMEM((CHUNK * C,), jnp.int32),
        pltpu.VMEM((CHUNK * HIST,), jnp.float32),
    ],
)(_hist_body)


_INV_2PI = 0.15915494309189535
_C_HI = 6.2831853
_C_LO = float(2.0 * np.pi - 6.2831853)
_S1 = 9.9999970687e-01
_S3 = -1.6666577176e-01
_S5 = 8.3325578492e-03
_S7 = -1.9812568137e-04
_S9 = 2.7040424852e-06
_S11 = -2.0533874769e-08


def _fast_sin(ang):
    k = jnp.floor(ang * _INV_2PI + 0.5)
    r = ang - k * _C_HI
    r = r - k * _C_LO
    y = r * r
    p = _S11
    for c in (_S9, _S7, _S5, _S3, _S1):
        p = p * y + c
    return p * r


def _assemble_body(s_ref, rp_ref, sm_ref, hist_ref,
                   rna_t_ref, mod_t_ref, combo_ref, fp_ref, out_ref):
    b = BLK
    s = s_ref[0, 0, :]
    sm = sm_ref[0, 0, :]
    pos = rp_ref[0, 0, :].astype(jnp.float32)

    iota8 = lax.broadcasted_iota(jnp.int32, (b, NUM_RNA_TYPE), 1)
    oh_s = (s[:, None] == iota8).astype(jnp.float32)
    rna = jnp.dot(oh_s, rna_t_ref[...], preferred_element_type=jnp.float32)
    iota3 = lax.broadcasted_iota(jnp.int32, (b, 3), 1)
    oh_m = (sm[:, None] == iota3).astype(jnp.float32)
    rna = rna + jnp.dot(oh_m, mod_t_ref[...], preferred_element_type=jnp.float32)
    ang = pos[:, None] * fp_ref[0:1, :] + fp_ref[1:2, :]
    rna = rna + _fast_sin(ang)

    pooled = jnp.dot(hist_ref[...], combo_ref[...],
                     preferred_element_type=jnp.float32)
    denom = pooled[:, ATOM_EMBED:ATOM_EMBED + 1]
    out_ref[:, 0:RNA_EMBED] = rna
    out_ref[:, RNA_EMBED:OUT_D] = (
        pooled[:, 0:ATOM_EMBED] * (1.0 / (denom + EPS)))


@jax.jit
def _run(S, RP, A, AP, SM, rna_table, atom_table, atom_pos_table, mod_table):
    hist = _hist_kernel(A.astype(jnp.int32).reshape(N * C),
                        AP.astype(jnp.int32).reshape(N * C)).reshape(N, HIST)

    combo = jnp.zeros((HIST, 128), jnp.float32)
    combo = combo.at[0:NUM_ATOM_TYPE, 0:ATOM_EMBED].set(atom_table)
    combo = combo.at[NUM_ATOM_TYPE:HIST, 0:ATOM_EMBED].set(atom_pos_table)
    combo = combo.at[0:NUM_ATOM_TYPE, ATOM_EMBED].set(1.0)

    d = np.arange(RNA_EMBED)
    freq_np = np.power(10000.0, -2.0 * (d // 2) / RNA_EMBED)
    phase_np = np.where(d % 2 == 0, 0.0, np.pi / 2)
    fp = jnp.asarray(np.stack([freq_np, phase_np]), jnp.float32)

    nb = N // BLK
    s3 = S.reshape(nb, 1, BLK).astype(jnp.int32)
    rp3 = RP.reshape(nb, 1, BLK).astype(jnp.int32)
    sm3 = SM.reshape(nb, 1, BLK).astype(jnp.int32)
    vec_spec = pl.BlockSpec((1, 1, BLK), lambda i: (i, 0, 0))

    return pl.pallas_call(
        _assemble_body,
        grid=(nb,),
        in_specs=[
            vec_spec, vec_spec, vec_spec,
            pl.BlockSpec((BLK, HIST), lambda i: (i, 0)),
            pl.BlockSpec((NUM_RNA_TYPE, RNA_EMBED), lambda i: (0, 0)),
            pl.BlockSpec((3, RNA_EMBED), lambda i: (0, 0)),
            pl.BlockSpec((HIST, 128), lambda i: (0, 0)),
            pl.BlockSpec((2, RNA_EMBED), lambda i: (0, 0)),
        ],
        out_specs=pl.BlockSpec((BLK, OUT_D), lambda i: (i, 0)),
        out_shape=jax.ShapeDtypeStruct((N, OUT_D), jnp.float32),
    )(s3, rp3, sm3, hist, rna_table, mod_table, combo, fp)


def kernel(S, RP, A, AP, SM, rna_table, atom_table, atom_pos_table, mod_table):
    return _run(S, RP, A, AP, SM, rna_table, atom_table,
                atom_pos_table, mod_table)


# speedup vs baseline: 1.3378x; 1.0065x over previous
import functools

import jax
import jax.numpy as jnp
import numpy as np
from jax import lax
from jax.experimental import pallas as pl
from jax.experimental.pallas import tpu as pltpu
from jax.experimental.pallas import tpu_sc as plsc

N = 16384
C = 16
RNA_EMBED = 128
ATOM_EMBED = 64
NUM_RNA_TYPE = 8
NUM_ATOM_TYPE = 64
NUM_ATOM_POS = 64
HIST = NUM_ATOM_TYPE + NUM_ATOM_POS
OUT_D = RNA_EMBED + ATOM_EMBED
EPS = 1e-10
BLK = 4096

NUM_SC = 2
NUM_SUBCORES = 16
NW = NUM_SC * NUM_SUBCORES
CHUNK = N // NW


def _hist_body(a_hbm, ap_hbm, hist_hbm, a_v, ap_v, hist_v):
    wid = lax.axis_index("s") * NUM_SC + lax.axis_index("c")
    base = wid * CHUNK
    pltpu.sync_copy(a_hbm.at[pl.ds(base * C, CHUNK * C)], a_v)
    pltpu.sync_copy(ap_hbm.at[pl.ds(base * C, CHUNK * C)], ap_v)

    def row(n, _):
        a_vec = a_v[pl.ds(n * C, 16)]
        ap_vec = ap_v[pl.ds(n * C, 16)]
        m = jnp.where(ap_vec != 0, 1.0, 0.0).astype(jnp.float32)
        for k in range(HIST // 16):
            hist_v[pl.ds(n * HIST + 16 * k, 16)] = jnp.zeros((16,), jnp.float32)
        flat_base = jnp.full((16,), n * HIST, jnp.int32)
        plsc.addupdate_scatter(hist_v, [flat_base + a_vec], m)
        plsc.addupdate_scatter(
            hist_v, [flat_base + (ap_vec + NUM_ATOM_TYPE)], m)
        return 0

    lax.fori_loop(0, CHUNK, row, 0)
    pltpu.sync_copy(hist_v, hist_hbm.at[pl.ds(base * HIST, CHUNK * HIST)])


_hist_kernel = functools.partial(
    pl.kernel,
    mesh=plsc.VectorSubcoreMesh(core_axis_name="c", subcore_axis_name="s"),
    out_type=jax.ShapeDtypeStruct((N * HIST,), jnp.float32),
    compiler_params=pltpu.CompilerParams(needs_layout_passes=False),
    scratch_types=[
        pltpu.VMEM((CHUNK * C,), jnp.int32),
        pltpu.VMEM((CHUNK * C,), jnp.int32),
        pltpu.VMEM((CHUNK * HIST,), jnp.float32),
    ],
)(_hist_body)


_INV_2PI = 0.15915494309189535
_C_HI = 6.2831853
_C_LO = float(2.0 * np.pi - 6.2831853)
_S1 = 9.9999970687e-01
_S3 = -1.6666577176e-01
_S5 = 8.3325578492e-03
_S7 = -1.9812568137e-04
_S9 = 2.7040424852e-06
_S11 = -2.0533874769e-08


def _fast_sin(ang):
    k = jnp.floor(ang * _INV_2PI + 0.5)
    r = ang - k * _C_HI
    r = r - k * _C_LO
    y = r * r
    p = _S11
    for c in (_S9, _S7, _S5, _S3, _S1):
        p = p * y + c
    return p * r


def _assemble_body(s_ref, rp_ref, sm_ref, hist_ref,
                   rna_t_ref, mod_t_ref, combo_ref, fp_ref, out_ref):
    b = BLK
    s = s_ref[0, 0, :]
    sm = sm_ref[0, 0, :]
    pos = rp_ref[0, 0, :].astype(jnp.float32)

    iota8 = lax.broadcasted_iota(jnp.int32, (b, NUM_RNA_TYPE), 1)
    oh_s = (s[:, None] == iota8).astype(jnp.float32)
    rna = jnp.dot(oh_s, rna_t_ref[...], preferred_element_type=jnp.float32)
    iota3 = lax.broadcasted_iota(jnp.int32, (b, 3), 1)
    oh_m = (sm[:, None] == iota3).astype(jnp.float32)
    rna = rna + jnp.dot(oh_m, mod_t_ref[...], preferred_element_type=jnp.float32)
    ang = pos[:, None] * fp_ref[0:1, :] + fp_ref[1:2, :]
    rna = rna + _fast_sin(ang)

    pooled = jnp.dot(hist_ref[...], combo_ref[...],
                     preferred_element_type=jnp.float32)
    denom = pooled[:, ATOM_EMBED:ATOM_EMBED + 1]
    out_ref[:, 0:RNA_EMBED] = rna
    out_ref[:, RNA_EMBED:OUT_D] = (
        pooled[:, 0:ATOM_EMBED] * (1.0 / (denom + EPS)))


@jax.jit
def _run(S, RP, A, AP, SM, rna_table, atom_table, atom_pos_table, mod_table):
    hist = _hist_kernel(A.astype(jnp.int32).reshape(N * C),
                        AP.astype(jnp.int32).reshape(N * C)).reshape(N, HIST)

    combo = jnp.zeros((HIST, 128), jnp.float32)
    combo = combo.at[0:NUM_ATOM_TYPE, 0:ATOM_EMBED].set(atom_table)
    combo = combo.at[NUM_ATOM_TYPE:HIST, 0:ATOM_EMBED].set(atom_pos_table)
    combo = combo.at[0:NUM_ATOM_TYPE, ATOM_EMBED].set(1.0)

    d = np.arange(RNA_EMBED)
    freq_np = np.power(10000.0, -2.0 * (d // 2) / RNA_EMBED)
    phase_np = np.where(d % 2 == 0, 0.0, np.pi / 2)
    fp = jnp.asarray(np.stack([freq_np, phase_np]), jnp.float32)

    nb = N // BLK
    s3 = S.reshape(nb, 1, BLK).astype(jnp.int32)
    rp3 = RP.reshape(nb, 1, BLK).astype(jnp.int32)
    sm3 = SM.reshape(nb, 1, BLK).astype(jnp.int32)
    vec_spec = pl.BlockSpec((1, 1, BLK), lambda i: (i, 0, 0))

    return pl.pallas_call(
        _assemble_body,
        grid=(nb,),
        in_specs=[
            vec_spec, vec_spec, vec_spec,
            pl.BlockSpec((BLK, HIST), lambda i: (i, 0)),
            pl.BlockSpec((NUM_RNA_TYPE, RNA_EMBED), lambda i: (0, 0)),
            pl.BlockSpec((3, RNA_EMBED), lambda i: (0, 0)),
            pl.BlockSpec((HIST, 128), lambda i: (0, 0)),
            pl.BlockSpec((2, RNA_EMBED), lambda i: (0, 0)),
        ],
        out_specs=pl.BlockSpec((BLK, OUT_D), lambda i: (i, 0)),
        out_shape=jax.ShapeDtypeStruct((N, OUT_D), jnp.float32),
    )(s3, rp3, sm3, hist, rna_table, mod_table, combo, fp)


def kernel(S, RP, A, AP, SM, rna_table, atom_table, atom_pos_table, mod_table):
    return _run(S, RP, A, AP, SM, rna_table, atom_table,
                atom_pos_table, mod_table)
